# Initial kernel scaffold; baseline (speedup 1.0000x reference)
import jax
import jax.numpy as jnp
from jax.experimental import pallas as pl


def kernel(x_miRNA, x_disease, edge_label_index):
    raise NotImplementedError("write your pallas kernel here")


# speedup vs baseline: 2.7843x; 2.7843x over previous
import jax
import jax.numpy as jnp
from jax import lax
from jax.experimental import pallas as pl
from jax.experimental.pallas import tpu as pltpu
from jax.experimental.pallas import tpu_sc as plsc

N_ROWS_TABLE = 10000
D = 128
E = 320000
CH = 128
NCHUNKS = E // CH
NC, NS, L = 2, 16, 16
NW = NC * NS
BASE_CHUNKS = NCHUNKS // NW
EXTRA = NCHUNKS % NW


def _edge_dot_kernel(xm, xd, srcr, dstr, out, idx_a, idx_b, a_rows, b_rows,
                     out_v, sem):
    wid = lax.axis_index("s") * NC + lax.axis_index("c")
    nchunks = jnp.where(wid < EXTRA, BASE_CHUNKS + 1, BASE_CHUNKS)
    lanes = lax.iota(jnp.int32, L)

    def chunk_body(i, carry):
        r = wid + i * NW
        pltpu.sync_copy(srcr.at[r], idx_a)
        pltpu.sync_copy(dstr.at[r], idx_b)
        ha = pltpu.async_copy(xm.at[idx_a], a_rows, sem)
        hb = pltpu.async_copy(xd.at[idx_b], b_rows, sem)
        ha.wait()
        hb.wait()

        def group_body(g, c2):
            vec = jnp.zeros((L,), jnp.float32)
            for l in range(L):
                e = g * L + l
                acc = a_rows[e, pl.ds(0, L)] * b_rows[e, pl.ds(0, L)]
                for k in range(1, D // L):
                    acc = acc + (a_rows[e, pl.ds(k * L, L)] *
                                 b_rows[e, pl.ds(k * L, L)])
                s = jnp.sum(acc)
                vec = jnp.where(lanes == l, s, vec)
            out_v[pl.ds(g * L, L)] = vec
            return c2

        lax.fori_loop(0, CH // L, group_body, 0, unroll=False)
        pltpu.sync_copy(out_v, out.at[pl.ds(r * CH, CH)])
        return carry

    lax.fori_loop(0, nchunks, chunk_body, 0, unroll=False)


def kernel(x_miRNA, x_disease, edge_label_index):
    eli = edge_label_index.astype(jnp.int32)
    src_r = eli[0].reshape(NCHUNKS, CH)
    dst_r = eli[1].reshape(NCHUNKS, CH)

    mesh = plsc.VectorSubcoreMesh(core_axis_name="c", subcore_axis_name="s")
    f = pl.kernel(
        _edge_dot_kernel,
        out_type=jax.ShapeDtypeStruct((E,), jnp.float32),
        mesh=mesh,
        scratch_types=[
            pltpu.VMEM((CH,), jnp.int32),
            pltpu.VMEM((CH,), jnp.int32),
            pltpu.VMEM((CH, D), jnp.float32),
            pltpu.VMEM((CH, D), jnp.float32),
            pltpu.VMEM((CH,), jnp.float32),
            pltpu.SemaphoreType.DMA,
        ],
        compiler_params=pltpu.CompilerParams(needs_layout_passes=False),
    )
    return f(x_miRNA, x_disease, src_r, dst_r)
